# baseline (speedup 1.0000x reference)
import functools

import jax
import jax.numpy as jnp
from jax import lax
from jax.experimental import pallas as pl
from jax.experimental.pallas import tpu as pltpu
from jax.experimental.pallas import tpu_sc as plsc

_NC = 2
_NS = 16
_NW = _NC * _NS
_CHUNK = 256
_PADW = 128
_NBUF = 4


@functools.lru_cache(maxsize=None)
def _build_gather(n_rows: int, emb_dim: int, n_chunks: int, n_tab: int):
    @functools.partial(
        pl.kernel,
        out_type=jax.ShapeDtypeStruct((n_rows, _PADW), jnp.float32),
        mesh=plsc.VectorSubcoreMesh(core_axis_name="c", subcore_axis_name="s"),
        scratch_types=[
            pltpu.VMEM((n_chunks, _CHUNK), jnp.int32),
            pltpu.VMEM((_NBUF, _CHUNK, emb_dim), jnp.float32),
            [pltpu.SemaphoreType.DMA] * _NBUF,
        ],
        compiler_params=pltpu.CompilerParams(use_tc_tiling_on_sc=False),
    )
    def gather_kernel(idx_hbm, table_hbm, out_hbm, idx_v, rows_v, sems):
        wid = lax.axis_index("s") * _NC + lax.axis_index("c")
        pltpu.sync_copy(idx_hbm.at[wid], idx_v)
        base = wid * (n_chunks * _CHUNK)

        @pl.loop(0, n_chunks, step=_NBUF)
        def _(c):
            copies = [
                pltpu.async_copy(
                    table_hbm.at[idx_v.at[c + j]], rows_v.at[j], sems[j]
                )
                for j in range(_NBUF)
            ]
            for j in range(_NBUF):
                copies[j].wait()
                pltpu.sync_copy(
                    rows_v.at[j],
                    out_hbm.at[
                        pl.ds(base + (c + j) * _CHUNK, _CHUNK), pl.ds(0, emb_dim)
                    ],
                )

    return gather_kernel


def kernel(item_ids, item_emb):
    batch, hist = item_ids.shape
    n_items, emb_dim = item_emb.shape
    n_rows = batch * hist
    assert n_rows % (_NW * _CHUNK) == 0
    n_chunks = n_rows // (_NW * _CHUNK)
    n_tab = (n_items + 7) // 8 * 8
    halves = n_tab * _PADW // emb_dim
    table = jnp.pad(item_emb, ((0, n_tab - n_items), (0, _PADW - emb_dim)))
    table = table.reshape(halves, emb_dim)
    scale = _PADW // emb_dim
    ids = (item_ids.astype(jnp.int32) * scale).reshape(_NW, n_chunks, _CHUNK)
    out = _build_gather(n_rows, emb_dim, n_chunks, n_tab)(ids, table)
    return out.reshape(batch, hist, _PADW)[:, :, :emb_dim]


# speedup vs baseline: 1.0113x; 1.0113x over previous
import functools

import jax
import jax.numpy as jnp
from jax import lax
from jax.experimental import pallas as pl
from jax.experimental.pallas import tpu as pltpu
from jax.experimental.pallas import tpu_sc as plsc

_NC = 2
_NS = 16
_NW = _NC * _NS
_CHUNK = 256
_PADW = 128
_NBUF = 4


@functools.lru_cache(maxsize=None)
def _build_gather(n_rows: int, emb_dim: int, n_chunks: int, n_tab: int):
    @functools.partial(
        pl.kernel,
        out_type=jax.ShapeDtypeStruct((n_rows, _PADW), jnp.float32),
        mesh=plsc.VectorSubcoreMesh(core_axis_name="c", subcore_axis_name="s"),
        scratch_types=[
            pltpu.VMEM((n_chunks, _CHUNK), jnp.int32),
            pltpu.VMEM((_NBUF, _CHUNK, emb_dim), jnp.float32),
            [pltpu.SemaphoreType.DMA] * _NBUF,
        ],
        compiler_params=pltpu.CompilerParams(use_tc_tiling_on_sc=False),
    )
    def gather_kernel(idx_hbm, table_hbm, out_hbm, idx_v, rows_v, sems):
        wid = lax.axis_index("s") * _NC + lax.axis_index("c")
        pltpu.sync_copy(idx_hbm.at[wid], idx_v)
        base = wid * (n_chunks * _CHUNK)
        half = _NBUF // 2

        def fire(c, s0):
            for j in range(half):
                pltpu.async_copy(
                    table_hbm.at[idx_v.at[c + j]], rows_v.at[s0 + j], sems[s0 + j]
                )

        def drain(c, s0):
            for j in range(half):
                pltpu.make_async_copy(
                    table_hbm.at[idx_v.at[c + j]], rows_v.at[s0 + j], sems[s0 + j]
                ).wait()
                pltpu.sync_copy(
                    rows_v.at[s0 + j],
                    out_hbm.at[
                        pl.ds(base + (c + j) * _CHUNK, _CHUNK), pl.ds(0, emb_dim)
                    ],
                )

        fire(0, 0)
        fire(half, half)

        @pl.loop(0, n_chunks - 2 * half, step=2 * half)
        def _(c):
            drain(c, 0)
            fire(c + 2 * half, 0)
            drain(c + half, half)
            fire(c + 3 * half, half)

        c_last = n_chunks - 2 * half
        drain(c_last, 0)
        drain(c_last + half, half)

    return gather_kernel


def kernel(item_ids, item_emb):
    batch, hist = item_ids.shape
    n_items, emb_dim = item_emb.shape
    n_rows = batch * hist
    assert n_rows % (_NW * _CHUNK) == 0
    n_chunks = n_rows // (_NW * _CHUNK)
    n_tab = (n_items + 7) // 8 * 8
    halves = n_tab * _PADW // emb_dim
    table = jnp.pad(item_emb, ((0, n_tab - n_items), (0, _PADW - emb_dim)))
    table = table.reshape(halves, emb_dim)
    scale = _PADW // emb_dim
    ids = (item_ids.astype(jnp.int32) * scale).reshape(_NW, n_chunks, _CHUNK)
    out = _build_gather(n_rows, emb_dim, n_chunks, n_tab)(ids, table)
    return out.reshape(batch, hist, _PADW)[:, :, :emb_dim]
